# baseline (speedup 1.0000x reference)
import functools

import jax
import jax.numpy as jnp
from jax import lax
from jax.experimental import pallas as pl
from jax.experimental.pallas import tpu as pltpu
from jax.experimental.pallas import tpu_sc as plsc

_VOCAB = 100000
_EMBD = 64
_BATCH = 4096
_MAX_TIME = 200

_NW = 32
_BLK = _BATCH // _NW


def _transpose_128_to(src, dst, e_rows):
    lane = jnp.arange(16, dtype=jnp.int32)
    bidxs = [lane + (g * 16) for g in range(8)]

    @plsc.parallel_loop(0, e_rows, unroll=8)
    def _row(e):
        eidx = jnp.zeros((16,), jnp.int32) + e
        for g in range(8):
            dst[e, pl.ds(g * 16, 16)] = plsc.load_gather(src, [bidxs[g], eidx])


def _body(inputs_t_hbm, lens_hbm, table_hbm, out_hbm, fs_hbm,
          idx_v, rows_v, tr_v, pidx0_v, pidx1_v, lens_v, ids_v,
          gsem0, gsem1, ssem0, ssem1, fsem):
    wid = lax.axis_index("s") * 2 + lax.axis_index("c")
    b0 = wid * _BLK
    gsems = (gsem0, gsem1)
    ssems = (ssem0, ssem1)

    pltpu.sync_copy(inputs_t_hbm.at[:, pl.ds(b0, _BLK)], idx_v)

    pidxs = (pidx0_v, pidx1_v)

    def start_gather(p, b):
        for t in range(2):
            for g in range(8):
                pidxs[b][pl.ds(t * _BLK + g * 16, 16)] = (
                    idx_v[2 * p + t, pl.ds(g * 16, 16)])
        pltpu.async_copy(
            table_hbm.at[pidxs[b]], rows_v.at[b], gsems[b])

    def wait_gather(b):
        pltpu.make_async_copy(
            table_hbm.at[pidxs[b]], rows_v.at[b], gsems[b]).wait()

    def start_scatter(p, q):
        pltpu.async_copy(
            tr_v.at[q], out_hbm.at[pl.ds(2 * p, 2), :, pl.ds(b0, _BLK)],
            ssems[q])

    def wait_scatter(q):
        pltpu.make_async_copy(
            tr_v.at[q], out_hbm.at[pl.ds(0, 2), :, pl.ds(b0, _BLK)],
            ssems[q]).wait()

    start_gather(0, 0)

    @pl.loop(0, _MAX_TIME // 4)
    def _p_pair(i):
        for q in range(2):
            p = i * 2 + q
            wait_gather(q)

            @pl.when(p + 1 < _MAX_TIME // 2)
            def _():
                start_gather(p + 1, 1 - q)

            @pl.when(p >= 2)
            def _():
                wait_scatter(q)

            _transpose_128_to(rows_v.at[q].at[pl.ds(0, _BLK)],
                              tr_v.at[q].at[0], _EMBD)
            _transpose_128_to(rows_v.at[q].at[pl.ds(_BLK, _BLK)],
                              tr_v.at[q].at[1], _EMBD)
            start_scatter(p, q)

    for q in range(2):
        wait_scatter(q)

    pltpu.sync_copy(lens_hbm.at[pl.ds(b0, _BLK)], lens_v)
    for i in range(_BLK // 16):
        lens = lens_v[pl.ds(i * 16, 16)]
        last = jnp.clip(lens - 1, 0, _MAX_TIME - 1)
        col = jnp.arange(16, dtype=jnp.int32) + (i * 16)
        ids_v[pl.ds(i * 16, 16)] = plsc.load_gather(idx_v, [last, col])
    pltpu.async_copy(
        table_hbm.at[ids_v], rows_v.at[0].at[pl.ds(0, _BLK)], fsem).wait()
    _transpose_128_to(rows_v.at[0].at[pl.ds(0, _BLK)],
                      tr_v.at[0].at[0], _EMBD)
    pltpu.sync_copy(tr_v.at[0].at[0], fs_hbm.at[:, pl.ds(b0, _BLK)])


@functools.cache
def _build():
    mesh = plsc.VectorSubcoreMesh(core_axis_name="c", subcore_axis_name="s")
    return pl.kernel(
        _body,
        out_type=(
            jax.ShapeDtypeStruct((_MAX_TIME, _EMBD, _BATCH), jnp.float32),
            jax.ShapeDtypeStruct((_EMBD, _BATCH), jnp.float32),
        ),
        mesh=mesh,
        scratch_types=[
            pltpu.VMEM((_MAX_TIME, _BLK), jnp.int32),
            pltpu.VMEM((2, 2 * _BLK, 128), jnp.float32),
            pltpu.VMEM((2, 2, _EMBD, _BLK), jnp.float32),
            pltpu.VMEM((2 * _BLK,), jnp.int32),
            pltpu.VMEM((2 * _BLK,), jnp.int32),
            pltpu.VMEM((_BLK,), jnp.int32),
            pltpu.VMEM((_BLK,), jnp.int32),
            pltpu.SemaphoreType.DMA,
            pltpu.SemaphoreType.DMA,
            pltpu.SemaphoreType.DMA,
            pltpu.SemaphoreType.DMA,
            pltpu.SemaphoreType.DMA,
        ],
        compiler_params=pltpu.CompilerParams(
            use_tc_tiling_on_sc=True, needs_layout_passes=False),
    )


def kernel(inputs, input_lengths, table):
    inputs_t = inputs.T
    table_p = jnp.concatenate(
        [table, jnp.zeros_like(table)], axis=1)
    out_t, fs_t = _build()(inputs_t, input_lengths, table_p)
    return out_t.transpose(2, 0, 1), fs_t.T


# speedup vs baseline: 1.0216x; 1.0216x over previous
import functools

import jax
import jax.numpy as jnp
from jax import lax
from jax.experimental import pallas as pl
from jax.experimental.pallas import tpu as pltpu
from jax.experimental.pallas import tpu_sc as plsc

_VOCAB = 100000
_EMBD = 64
_BATCH = 4096
_MAX_TIME = 200

_NW = 32
_B_PER_W = _BATCH // _NW
_IDX_PER_W = _B_PER_W * _MAX_TIME
_CHUNK = 640
_NCHUNK = _IDX_PER_W // _CHUNK


def _body(inputs_hbm, lens_hbm, table_hbm, enc_hbm, fs_hbm,
          idx_v, rows_v, lens_v, pos_v, ids_v, fs_v,
          gsem0, gsem1, fsem):
    wid = lax.axis_index("s") * 2 + lax.axis_index("c")
    ibase = wid * _IDX_PER_W
    bbase = wid * _B_PER_W
    gsems = (gsem0, gsem1)

    pltpu.sync_copy(inputs_hbm.at[pl.ds(ibase, _IDX_PER_W)], idx_v)

    def start_gather(g, b):
        pltpu.async_copy(
            table_hbm.at[idx_v.at[pl.ds(g * _CHUNK, _CHUNK)]],
            rows_v.at[b], gsems[b])

    def wait_gather(b):
        pltpu.make_async_copy(
            table_hbm.at[idx_v.at[pl.ds(0, _CHUNK)]],
            rows_v.at[b], gsems[b]).wait()

    start_gather(0, 0)

    @pl.loop(0, _NCHUNK // 2)
    def _chunk_pair(i):
        for b in range(2):
            g = i * 2 + b
            wait_gather(b)

            @pl.when(g + 1 < _NCHUNK)
            def _():
                start_gather(g + 1, 1 - b)

            pltpu.sync_copy(rows_v.at[b],
                            enc_hbm.at[pl.ds(ibase + g * _CHUNK, _CHUNK)])

    pltpu.sync_copy(lens_hbm.at[pl.ds(bbase, _B_PER_W)], lens_v)
    for i in range(_B_PER_W // 16):
        lens = lens_v[pl.ds(i * 16, 16)]
        last = jnp.clip(lens - 1, 0, _MAX_TIME - 1)
        b_abs = jnp.arange(16, dtype=jnp.int32) + (bbase + i * 16)
        pos_v[pl.ds(i * 16, 16)] = b_abs * _MAX_TIME + last
    pltpu.async_copy(inputs_hbm.at[pos_v], ids_v, fsem).wait()
    pltpu.async_copy(table_hbm.at[ids_v], fs_v, fsem).wait()
    pltpu.sync_copy(fs_v, fs_hbm.at[pl.ds(bbase, _B_PER_W)])


@functools.cache
def _build():
    mesh = plsc.VectorSubcoreMesh(core_axis_name="c", subcore_axis_name="s")
    return pl.kernel(
        _body,
        out_type=(
            jax.ShapeDtypeStruct((_BATCH * _MAX_TIME, _EMBD), jnp.float32),
            jax.ShapeDtypeStruct((_BATCH, _EMBD), jnp.float32),
        ),
        mesh=mesh,
        scratch_types=[
            pltpu.VMEM((_IDX_PER_W,), jnp.int32),
            pltpu.VMEM((2, _CHUNK, _EMBD), jnp.float32),
            pltpu.VMEM((_B_PER_W,), jnp.int32),
            pltpu.VMEM((_B_PER_W,), jnp.int32),
            pltpu.VMEM((_B_PER_W,), jnp.int32),
            pltpu.VMEM((_B_PER_W, _EMBD), jnp.float32),
            pltpu.SemaphoreType.DMA,
            pltpu.SemaphoreType.DMA,
            pltpu.SemaphoreType.DMA,
        ],
        compiler_params=pltpu.CompilerParams(use_tc_tiling_on_sc=False),
    )


def kernel(inputs, input_lengths, table):
    enc_flat, final_state = _build()(
        inputs.reshape(-1), input_lengths, table)
    return enc_flat.reshape(_BATCH, _MAX_TIME, _EMBD), final_state
